# baseline (speedup 1.0000x reference)
import functools

import jax
import jax.numpy as jnp
from jax import lax
from jax.experimental import pallas as pl
from jax.experimental.pallas import tpu as pltpu
from jax.experimental.pallas import tpu_sc as plsc

N = 10000
E = 320000
D = 128
NC = 2
NS = 16
NW = NC * NS
EPW = E // NW
CH = 40
NCHUNK = EPW // CH
BODY = 50
NBODY = NCHUNK // BODY
RPT = 624
REM = N - NS * RPT


def _sc_agg_body(with_cnt, *refs):
    if with_cnt:
        (table, src_h, dst_h, z, out, cnt_out,
         acc, src_b, dst_b, rows, cnt_v, gsem, ssem, isem) = refs
    else:
        (table, src_h, dst_h, z, out,
         acc, src_b, dst_b, rows, gsem, ssem, isem) = refs
    c = lax.axis_index("c")
    s = lax.axis_index("s")
    wid = c * NS + s

    pltpu.sync_copy(z, acc.at[pl.ds(s * RPT, RPT)])

    @pl.when(s == NS - 1)
    def _():
        pltpu.sync_copy(z.at[pl.ds(0, REM)], acc.at[pl.ds(NS * RPT, REM)])

    def load_idx(j, p):
        pltpu.async_copy(src_h.at[wid, pl.ds(j * BODY, BODY)], src_b.at[p],
                         isem)
        pltpu.async_copy(dst_h.at[wid, pl.ds(j * BODY, BODY)], dst_b.at[p],
                         isem)

    def wait_idx(j, p):
        pltpu.make_async_copy(src_h.at[wid, pl.ds(j * BODY, BODY)],
                              src_b.at[p], isem).wait()
        pltpu.make_async_copy(dst_h.at[wid, pl.ds(j * BODY, BODY)],
                              dst_b.at[p], isem).wait()

    load_idx(0, 0)

    if with_cnt:
        ones = jnp.ones((16,), jnp.float32)
        zeros = jnp.zeros((16,), jnp.float32)

        def zero_body(i, carry):
            cnt_v[pl.ds(i * 16, 16)] = zeros
            return carry
        lax.fori_loop(0, N // 16, zero_body, 0)

    plsc.subcore_barrier()

    nbuf = 6 if with_cnt else 8
    lead = nbuf // 2

    if with_cnt:
        lane = lax.iota(jnp.int32, 16)
        tail_mask = lane >= 8

        def count_chunk(p, t):
            idx = dst_b[p, t, pl.ds(0, 16)]
            plsc.addupdate_scatter(cnt_v, [idx], ones)
            idx = dst_b[p, t, pl.ds(16, 16)]
            plsc.addupdate_scatter(cnt_v, [idx], ones)
            idx = dst_b[p, t, pl.ds(24, 16)]
            plsc.addupdate_scatter(cnt_v, [idx], ones, mask=tail_mask)

    def pipe_body(j, carry):
        p = lax.rem(j, 2)
        wait_idx(j, p)

        @pl.when(j + 1 < NBODY)
        def _():
            load_idx(j + 1, 1 - p)

        def gather(t, b):
            return pltpu.async_copy(table.at[src_b.at[p, t]], rows.at[b],
                                    gsem)

        def scatter(t, b):
            return pltpu.async_copy(rows.at[b], acc.at[dst_b.at[p, t]], ssem,
                                    add=True)
        d = {}
        sc = {}
        for k in range(lead):
            d[k] = gather(k, k % nbuf)
        for t in range(BODY):
            if t - lead >= 0:
                sc[t - lead].wait()
            if t + lead < BODY:
                d[t + lead] = gather(t + lead, (t + lead) % nbuf)
            if with_cnt:
                count_chunk(p, t)
            d[t].wait()
            sc[t] = scatter(t, t % nbuf)
        for t in range(BODY - lead, BODY):
            sc[t].wait()
        return carry
    lax.fori_loop(0, NBODY, pipe_body, 0)

    plsc.subcore_barrier()
    pltpu.sync_copy(acc.at[pl.ds(s * RPT, RPT)], out.at[c, pl.ds(s * RPT, RPT)])

    @pl.when(s == NS - 1)
    def _():
        pltpu.sync_copy(acc.at[pl.ds(NS * RPT, REM)],
                        out.at[c, pl.ds(NS * RPT, REM)])

    if with_cnt:
        pltpu.sync_copy(cnt_v, cnt_out.at[wid, 0])


def _make_sc_agg(with_cnt):
    mesh = plsc.VectorSubcoreMesh(core_axis_name="c", subcore_axis_name="s")
    out_type = [jax.ShapeDtypeStruct((NC, N, D), jnp.float32)]
    if with_cnt:
        out_type.append(jax.ShapeDtypeStruct((NW, 1, N), jnp.float32))
    scratch = [
        pltpu.VMEM_SHARED((N, D), jnp.float32),
        pltpu.VMEM((2, BODY, CH), jnp.int32),
        pltpu.VMEM((2, BODY, CH), jnp.int32),
        pltpu.VMEM(((6 if with_cnt else 8), CH, D), jnp.float32),
    ]
    if with_cnt:
        scratch.append(pltpu.VMEM((N,), jnp.float32))
    scratch.append(pltpu.SemaphoreType.DMA)
    scratch.append(pltpu.SemaphoreType.DMA)
    scratch.append(pltpu.SemaphoreType.DMA)
    return pl.kernel(
        functools.partial(_sc_agg_body, with_cnt),
        out_type=tuple(out_type),
        mesh=mesh,
        scratch_types=scratch,
        compiler_params=pltpu.CompilerParams(needs_layout_passes=False, use_tc_tiling_on_sc=False),
    )


def _dense_hidden_body(sp, cp, xr, x, wl, wrn, bn, g, be, out, xrn):
    agg = sp[0] + sp[1]
    cnt = jnp.sum(cp[...], axis=(0, 1))
    inv = 1.0 / jnp.maximum(cnt, 1.0)
    mean = agg * inv[:, None]
    y = jnp.dot(mean, wl[...], preferred_element_type=jnp.float32) + xr[...]
    m = jnp.mean(y, axis=0)
    v = jnp.mean((y - m) ** 2, axis=0)
    h = g[...] * (y - m) / jnp.sqrt(v + 1e-5) + be[...]
    h = jnp.maximum(h, 0.0)
    out[...] = h
    xrn[...] = (jnp.dot(h, wrn[...], preferred_element_type=jnp.float32)
                + bn[...])


def _dense_first_body(sp, cp, x, wl, wr, b, wrn, bn, g, be, out, xrn):
    agg = sp[0] + sp[1]
    cnt = jnp.sum(cp[...], axis=(0, 1))
    inv = 1.0 / jnp.maximum(cnt, 1.0)
    mean = agg * inv[:, None]
    y = (jnp.dot(mean, wl[...], preferred_element_type=jnp.float32)
         + jnp.dot(x[...], wr[...], preferred_element_type=jnp.float32)
         + b[...])
    m = jnp.mean(y, axis=0)
    v = jnp.mean((y - m) ** 2, axis=0)
    h = g[...] * (y - m) / jnp.sqrt(v + 1e-5) + be[...]
    h = jnp.maximum(h, 0.0)
    out[...] = h
    xrn[...] = (jnp.dot(h, wrn[...], preferred_element_type=jnp.float32)
                + bn[...])


def _dense_out_body(sp, cp, xr2, x, wl2, wl3, wr3, b3, mu, lv):
    agg = sp[0] + sp[1]
    cnt = jnp.sum(cp[...], axis=(0, 1))
    inv = 1.0 / jnp.maximum(cnt, 1.0)
    mean = agg * inv[:, None]
    y2 = jnp.dot(mean, wl2[...], preferred_element_type=jnp.float32) + xr2[...]
    y3 = (jnp.dot(mean, wl3[...], preferred_element_type=jnp.float32)
          + jnp.dot(x[...], wr3[...], preferred_element_type=jnp.float32)
          + b3[...])
    mu[...] = 1.0 / (1.0 + jnp.exp(-y2))
    lv[...] = 1.0 / (1.0 + jnp.exp(-y3))


_dense_first = pl.pallas_call(
    _dense_first_body,
    out_shape=(jax.ShapeDtypeStruct((N, D), jnp.float32),
               jax.ShapeDtypeStruct((N, D), jnp.float32)),
)

_dense_hidden = pl.pallas_call(
    _dense_hidden_body,
    out_shape=(jax.ShapeDtypeStruct((N, D), jnp.float32),
               jax.ShapeDtypeStruct((N, D), jnp.float32)),
)

_dense_out = pl.pallas_call(
    _dense_out_body,
    out_shape=(jax.ShapeDtypeStruct((N, D), jnp.float32),
               jax.ShapeDtypeStruct((N, D), jnp.float32)),
)


def kernel(x, edge_index, Wl0, Wr0, b0, Wl1, Wr1, b1, Wl2, Wr2, b2,
           Wl3, Wr3, b3, g0, be0, g1, be1):
    src = edge_index[0].reshape(NW, NCHUNK, CH)
    dst = edge_index[1].reshape(NW, NCHUNK, CH)
    z = jnp.zeros((RPT, D), jnp.float32)
    b0r = b0.reshape(1, D)
    b1r = b1.reshape(1, D)
    b2r = b2.reshape(1, D)
    b3r = b3.reshape(1, D)
    g0r = g0.reshape(1, D)
    be0r = be0.reshape(1, D)
    g1r = g1.reshape(1, D)
    be1r = be1.reshape(1, D)

    agg_cnt = _make_sc_agg(True)
    agg_only = _make_sc_agg(False)

    s0, cnt_parts = agg_cnt(x, src, dst, z)
    h1, xr1 = _dense_first(s0, cnt_parts, x, Wl0, Wr0, b0r, Wr1, b1r,
                           g0r, be0r)
    (s1,) = agg_only(h1, src, dst, z)
    h2, xr2 = _dense_hidden(s1, cnt_parts, xr1, h1, Wl1, Wr2, b2r,
                            g1r, be1r)
    (s2,) = agg_only(h2, src, dst, z)
    mu, lv = _dense_out(s2, cnt_parts, xr2, h2, Wl2, Wl3, Wr3, b3r)
    return (mu, lv)


# speedup vs baseline: 1.0239x; 1.0239x over previous
import functools

import jax
import jax.numpy as jnp
from jax import lax
from jax.experimental import pallas as pl
from jax.experimental.pallas import tpu as pltpu
from jax.experimental.pallas import tpu_sc as plsc

N = 10000
E = 320000
D = 128
NC = 2
NS = 16
NW = NC * NS
EPW = E // NW
CH = 40
NCHUNK = EPW // CH
BODY = 50
NBODY = NCHUNK // BODY
RPT = 624
REM = N - NS * RPT


def _sc_agg_body(with_cnt, *refs):
    if with_cnt:
        (table, src_h, dst_h, z, out, cnt_out,
         acc, src_b, dst_b, rows, cnt_v, gsem, ssem, isem) = refs
    else:
        (table, src_h, dst_h, z, out,
         acc, src_b, dst_b, rows, gsem, ssem, isem) = refs
    c = lax.axis_index("c")
    s = lax.axis_index("s")
    wid = c * NS + s

    pltpu.sync_copy(z, acc.at[pl.ds(s * RPT, RPT)])

    @pl.when(s == NS - 1)
    def _():
        pltpu.sync_copy(z.at[pl.ds(0, REM)], acc.at[pl.ds(NS * RPT, REM)])

    def load_idx(j, p):
        pltpu.async_copy(src_h.at[wid, pl.ds(j * BODY, BODY)], src_b.at[p],
                         isem)
        pltpu.async_copy(dst_h.at[wid, pl.ds(j * BODY, BODY)], dst_b.at[p],
                         isem)

    def wait_idx(j, p):
        pltpu.make_async_copy(src_h.at[wid, pl.ds(j * BODY, BODY)],
                              src_b.at[p], isem).wait()
        pltpu.make_async_copy(dst_h.at[wid, pl.ds(j * BODY, BODY)],
                              dst_b.at[p], isem).wait()

    load_idx(0, 0)

    if with_cnt:
        ones = jnp.ones((16,), jnp.float32)
        zeros = jnp.zeros((16,), jnp.float32)

        def zero_body(i, carry):
            cnt_v[pl.ds(i * 16, 16)] = zeros
            return carry
        lax.fori_loop(0, N // 16, zero_body, 0)

    plsc.subcore_barrier()

    nbuf = 6 if with_cnt else 8
    lead = nbuf // 2

    if with_cnt:
        lane = lax.iota(jnp.int32, 16)
        tail_mask = lane >= 8

        def count_chunk(p, t):
            idx = dst_b[p, t, pl.ds(0, 16)]
            plsc.addupdate_scatter(cnt_v, [idx], ones)
            idx = dst_b[p, t, pl.ds(16, 16)]
            plsc.addupdate_scatter(cnt_v, [idx], ones)
            idx = dst_b[p, t, pl.ds(24, 16)]
            plsc.addupdate_scatter(cnt_v, [idx], ones, mask=tail_mask)

    def pipe_body(j, carry):
        p = lax.rem(j, 2)
        wait_idx(j, p)

        @pl.when(j + 1 < NBODY)
        def _():
            load_idx(j + 1, 1 - p)

        def gather(t, b):
            return pltpu.async_copy(table.at[src_b.at[p, t]], rows.at[b],
                                    gsem)

        def scatter(t, b):
            return pltpu.async_copy(rows.at[b], acc.at[dst_b.at[p, t]], ssem,
                                    add=True)
        d = {}
        sc = {}
        for k in range(lead):
            d[k] = gather(k, k % nbuf)
        for t in range(BODY):
            if t - lead >= 0:
                sc[t - lead].wait()
            if t + lead < BODY:
                d[t + lead] = gather(t + lead, (t + lead) % nbuf)
            if with_cnt:
                count_chunk(p, t)
            d[t].wait()
            sc[t] = scatter(t, t % nbuf)
        for t in range(BODY - lead, BODY):
            sc[t].wait()
        return carry
    lax.fori_loop(0, NBODY, pipe_body, 0)

    plsc.subcore_barrier()
    pltpu.sync_copy(acc.at[pl.ds(s * RPT, RPT)], out.at[c, pl.ds(s * RPT, RPT)])

    @pl.when(s == NS - 1)
    def _():
        pltpu.sync_copy(acc.at[pl.ds(NS * RPT, REM)],
                        out.at[c, pl.ds(NS * RPT, REM)])

    if with_cnt:
        pltpu.sync_copy(cnt_v, cnt_out.at[wid, 0])


def _make_sc_agg(with_cnt):
    mesh = plsc.VectorSubcoreMesh(core_axis_name="c", subcore_axis_name="s")
    out_type = [jax.ShapeDtypeStruct((NC, N, D), jnp.float32)]
    if with_cnt:
        out_type.append(jax.ShapeDtypeStruct((NW, 1, N), jnp.float32))
    scratch = [
        pltpu.VMEM_SHARED((N, D), jnp.float32),
        pltpu.VMEM((2, BODY, CH), jnp.int32),
        pltpu.VMEM((2, BODY, CH), jnp.int32),
        pltpu.VMEM(((6 if with_cnt else 8), CH, D), jnp.float32),
    ]
    if with_cnt:
        scratch.append(pltpu.VMEM((N,), jnp.float32))
    scratch.append(pltpu.SemaphoreType.DMA)
    scratch.append(pltpu.SemaphoreType.DMA)
    scratch.append(pltpu.SemaphoreType.DMA)
    return pl.kernel(
        functools.partial(_sc_agg_body, with_cnt),
        out_type=tuple(out_type),
        mesh=mesh,
        scratch_types=scratch,
        compiler_params=pltpu.CompilerParams(needs_layout_passes=False, use_tc_tiling_on_sc=False),
    )


def _dense_hidden_body(sp, cp, x, wl, wr, b, g, be, out):
    agg = sp[0] + sp[1]
    cnt = jnp.sum(cp[...], axis=(0, 1))
    inv = 1.0 / jnp.maximum(cnt, 1.0)
    mean = agg * inv[:, None]
    y = (jnp.dot(mean, wl[...], preferred_element_type=jnp.float32)
         + jnp.dot(x[...], wr[...], preferred_element_type=jnp.float32)
         + b[...])
    m = jnp.mean(y, axis=0)
    v = jnp.mean((y - m) ** 2, axis=0)
    h = g[...] * (y - m) / jnp.sqrt(v + 1e-5) + be[...]
    out[...] = jnp.maximum(h, 0.0)


def _dense_out_body(sp, cp, x, wl2, wr2, b2, wl3, wr3, b3, mu, lv):
    agg = sp[0] + sp[1]
    cnt = jnp.sum(cp[...], axis=(0, 1))
    inv = 1.0 / jnp.maximum(cnt, 1.0)
    mean = agg * inv[:, None]
    y2 = (jnp.dot(mean, wl2[...], preferred_element_type=jnp.float32)
          + jnp.dot(x[...], wr2[...], preferred_element_type=jnp.float32)
          + b2[...])
    y3 = (jnp.dot(mean, wl3[...], preferred_element_type=jnp.float32)
          + jnp.dot(x[...], wr3[...], preferred_element_type=jnp.float32)
          + b3[...])
    mu[...] = 1.0 / (1.0 + jnp.exp(-y2))
    lv[...] = 1.0 / (1.0 + jnp.exp(-y3))


_dense_hidden = pl.pallas_call(
    _dense_hidden_body,
    out_shape=jax.ShapeDtypeStruct((N, D), jnp.float32),
)

_dense_out = pl.pallas_call(
    _dense_out_body,
    out_shape=(jax.ShapeDtypeStruct((N, D), jnp.float32),
               jax.ShapeDtypeStruct((N, D), jnp.float32)),
)


def kernel(x, edge_index, Wl0, Wr0, b0, Wl1, Wr1, b1, Wl2, Wr2, b2,
           Wl3, Wr3, b3, g0, be0, g1, be1):
    src = edge_index[0].reshape(NW, NCHUNK, CH)
    dst = edge_index[1].reshape(NW, NCHUNK, CH)
    z = jnp.zeros((RPT, D), jnp.float32)
    b0r = b0.reshape(1, D)
    b1r = b1.reshape(1, D)
    b2r = b2.reshape(1, D)
    b3r = b3.reshape(1, D)
    g0r = g0.reshape(1, D)
    be0r = be0.reshape(1, D)
    g1r = g1.reshape(1, D)
    be1r = be1.reshape(1, D)

    agg_cnt = _make_sc_agg(True)
    agg_only = _make_sc_agg(False)

    s0, cnt_parts = agg_cnt(x, src, dst, z)
    h1 = _dense_hidden(s0, cnt_parts, x, Wl0, Wr0, b0r, g0r, be0r)
    (s1,) = agg_only(h1, src, dst, z)
    h2 = _dense_hidden(s1, cnt_parts, h1, Wl1, Wr1, b1r, g1r, be1r)
    (s2,) = agg_only(h2, src, dst, z)
    mu, lv = _dense_out(s2, cnt_parts, h2, Wl2, Wr2, b2r, Wl3, Wr3, b3r)
    return (mu, lv)
